# Initial kernel scaffold; baseline (speedup 1.0000x reference)
import jax
import jax.numpy as jnp
from jax.experimental import pallas as pl


def kernel(indices, weight):
    raise NotImplementedError("write your pallas kernel here")


# speedup vs baseline: 2.7525x; 2.7525x over previous
import functools

import jax
import jax.numpy as jnp
from jax import lax
from jax.experimental import pallas as pl
from jax.experimental.pallas import tpu as pltpu
from jax.experimental.pallas import tpu_sc as plsc

NUM_CORES = 2
NUM_SUBCORES = 16
NUM_WORKERS = NUM_CORES * NUM_SUBCORES
CHUNK = 128


def _gather_body(idx_hbm, table_hbm, out_hbm, idx_v, rows_v, sem):
    n = idx_hbm.shape[0]
    n_per_w = n // NUM_WORKERS
    n_chunks = n_per_w // CHUNK
    wid = lax.axis_index("s") * NUM_CORES + lax.axis_index("c")
    base = wid * n_per_w

    def body(i, _):
        off = pl.multiple_of(base + i * CHUNK, CHUNK)
        pltpu.sync_copy(idx_hbm.at[pl.ds(off, CHUNK)], idx_v)
        pltpu.async_copy(table_hbm.at[idx_v], rows_v, sem).wait()
        pltpu.sync_copy(rows_v, out_hbm.at[pl.ds(off, CHUNK)])
        return ()

    lax.fori_loop(0, n_chunks, body, ())


def kernel(indices, weight):
    b, s = indices.shape
    v, d = weight.shape
    n = b * s
    flat_idx = indices.reshape(n).astype(jnp.int32)

    mesh = plsc.VectorSubcoreMesh(
        core_axis_name="c", subcore_axis_name="s",
        num_cores=NUM_CORES, num_subcores=NUM_SUBCORES,
    )
    run = functools.partial(
        pl.kernel,
        out_type=jax.ShapeDtypeStruct((n, d), jnp.float32),
        mesh=mesh,
        scratch_types=[
            pltpu.VMEM((CHUNK,), jnp.int32),
            pltpu.VMEM((CHUNK, d), jnp.float32),
            pltpu.SemaphoreType.DMA,
        ],
    )(_gather_body)
    out = run(flat_idx, weight)
    return out.reshape(b, s, d)


# speedup vs baseline: 3.3267x; 1.2086x over previous
import functools

import jax
import jax.numpy as jnp
from jax import lax
from jax.experimental import pallas as pl
from jax.experimental.pallas import tpu as pltpu
from jax.experimental.pallas import tpu_sc as plsc

NUM_CORES = 2
NUM_SUBCORES = 16
NUM_WORKERS = NUM_CORES * NUM_SUBCORES
CHUNK = 128


def _gather_body(idx_hbm, table_hbm, out_hbm, idx_all, rows0, rows1,
                 g0, g1, o0, o1):
    n = idx_hbm.shape[0]
    n_per_w = n // NUM_WORKERS
    n_chunks = n_per_w // CHUNK
    wid = lax.axis_index("s") * NUM_CORES + lax.axis_index("c")
    base = wid * n_per_w

    rows = (rows0, rows1)
    gsem = (g0, g1)
    osem = (o0, o1)

    pltpu.sync_copy(idx_hbm.at[pl.ds(base, n_per_w)], idx_all)

    def gather_start(i, b):
        pltpu.async_copy(
            table_hbm.at[idx_all.at[pl.ds(i * CHUNK, CHUNK)]],
            rows[b], gsem[b])

    def gather_wait(b):
        pltpu.make_async_copy(
            table_hbm.at[idx_all.at[pl.ds(0, CHUNK)]],
            rows[b], gsem[b]).wait()

    def out_start(i, b):
        pltpu.async_copy(
            rows[b], out_hbm.at[pl.ds(base + i * CHUNK, CHUNK)], osem[b])

    def out_wait(b):
        pltpu.make_async_copy(
            rows[b], out_hbm.at[pl.ds(base, CHUNK)], osem[b]).wait()

    gather_start(0, 0)
    gather_start(1, 1)
    gather_wait(0)
    out_start(0, 0)

    def pair(k, _):
        for b in (0, 1):
            i = 2 + 2 * k + b
            out_wait(b)
            gather_start(i, b)
            gather_wait(1 - b)
            out_start(i - 1, 1 - b)
        return ()

    lax.fori_loop(0, (n_chunks - 2) // 2, pair, (), unroll=False)

    gather_wait(1)
    out_start(n_chunks - 1, 1)
    out_wait(0)
    out_wait(1)


def kernel(indices, weight):
    b, s = indices.shape
    v, d = weight.shape
    n = b * s
    flat_idx = indices.reshape(n).astype(jnp.int32)

    mesh = plsc.VectorSubcoreMesh(
        core_axis_name="c", subcore_axis_name="s",
        num_cores=NUM_CORES, num_subcores=NUM_SUBCORES,
    )
    n_per_w = n // NUM_WORKERS
    run = functools.partial(
        pl.kernel,
        out_type=jax.ShapeDtypeStruct((n, d), jnp.float32),
        mesh=mesh,
        scratch_types=[
            pltpu.VMEM((n_per_w,), jnp.int32),
            pltpu.VMEM((CHUNK, d), jnp.float32),
            pltpu.VMEM((CHUNK, d), jnp.float32),
            pltpu.SemaphoreType.DMA,
            pltpu.SemaphoreType.DMA,
            pltpu.SemaphoreType.DMA,
            pltpu.SemaphoreType.DMA,
        ],
    )(_gather_body)
    out = run(flat_idx, weight)
    return out.reshape(b, s, d)


# speedup vs baseline: 5.9218x; 1.7801x over previous
import functools

import jax
import jax.numpy as jnp
from jax import lax
from jax.experimental import pallas as pl
from jax.experimental.pallas import tpu as pltpu
from jax.experimental.pallas import tpu_sc as plsc

NUM_CORES = 2
NUM_SUBCORES = 16
NUM_WORKERS = NUM_CORES * NUM_SUBCORES
SEQ = 50
SEQ_PAD = 56
BPB = 8


def _gather_body(idx_hbm, table_hbm, out_hbm, idx_all, rows0, rows1,
                 g0, g1, o0, o1):
    nb = out_hbm.shape[0]
    nb_per_w = nb // NUM_WORKERS
    n_slots = nb_per_w // BPB
    wid = lax.axis_index("s") * NUM_CORES + lax.axis_index("c")
    bbase = wid * nb_per_w

    rows = (rows0, rows1)
    gsem = (g0, g1)
    osem = (o0, o1)

    pltpu.sync_copy(
        idx_hbm.at[pl.ds(bbase * SEQ_PAD, nb_per_w * SEQ_PAD)], idx_all)

    def gathers_start(i, b):
        for j in range(BPB):
            pltpu.async_copy(
                table_hbm.at[idx_all.at[pl.ds((i * BPB + j) * SEQ_PAD, SEQ)]],
                rows[b].at[j], gsem[b])

    def gathers_wait(b):
        for j in range(BPB):
            pltpu.make_async_copy(
                table_hbm.at[idx_all.at[pl.ds(0, SEQ)]],
                rows[b].at[j], gsem[b]).wait()

    def out_start(i, b):
        pltpu.async_copy(
            rows[b], out_hbm.at[pl.ds(bbase + i * BPB, BPB)], osem[b])

    def out_wait(b):
        pltpu.make_async_copy(
            rows[b], out_hbm.at[pl.ds(bbase, BPB)], osem[b]).wait()

    gathers_start(0, 0)
    gathers_start(1, 1)
    gathers_wait(0)
    out_start(0, 0)

    def pair(k, _):
        for b in (0, 1):
            i = 2 + 2 * k + b
            out_wait(b)
            gathers_start(i, b)
            gathers_wait(1 - b)
            out_start(i - 1, 1 - b)
        return ()

    lax.fori_loop(0, (n_slots - 2) // 2, pair, (), unroll=False)

    gathers_wait(1)
    out_start(n_slots - 1, 1)
    out_wait(0)
    out_wait(1)


def kernel(indices, weight):
    b, s = indices.shape
    v, d = weight.shape
    idx_pad = jnp.pad(indices.astype(jnp.int32), ((0, 0), (0, SEQ_PAD - s)))
    flat_idx = idx_pad.reshape(b * SEQ_PAD)

    mesh = plsc.VectorSubcoreMesh(
        core_axis_name="c", subcore_axis_name="s",
        num_cores=NUM_CORES, num_subcores=NUM_SUBCORES,
    )
    nb_per_w = b // NUM_WORKERS
    run = functools.partial(
        pl.kernel,
        out_type=jax.ShapeDtypeStruct((b, s, d), jnp.float32),
        mesh=mesh,
        scratch_types=[
            pltpu.VMEM((nb_per_w * SEQ_PAD,), jnp.int32),
            pltpu.VMEM((BPB, s, d), jnp.float32),
            pltpu.VMEM((BPB, s, d), jnp.float32),
            pltpu.SemaphoreType.DMA,
            pltpu.SemaphoreType.DMA,
            pltpu.SemaphoreType.DMA,
            pltpu.SemaphoreType.DMA,
        ],
        compiler_params=pltpu.CompilerParams(use_tc_tiling_on_sc=True),
    )(_gather_body)
    return run(flat_idx, weight)


# speedup vs baseline: 10.4099x; 1.7579x over previous
import functools

import jax
import jax.numpy as jnp
from jax import lax
from jax.experimental import pallas as pl
from jax.experimental.pallas import tpu as pltpu
from jax.experimental.pallas import tpu_sc as plsc

NUM_CORES = 2
NUM_SUBCORES = 16
NUM_WORKERS = NUM_CORES * NUM_SUBCORES
BLOCK = 128


def _gather_body(idx_hbm, table_hbm, out_hbm, idx_all, rows0, rows1,
                 g0, g1, o0, o1):
    n_chunks = idx_hbm.shape[1]
    wid = lax.axis_index("s") * NUM_CORES + lax.axis_index("c")
    nbatch = out_hbm.shape[0] // n_chunks
    obase = wid * BLOCK

    rows = (rows0, rows1)
    gsem = (g0, g1)
    osem = (o0, o1)

    pltpu.sync_copy(idx_hbm.at[wid], idx_all)

    def gather_start(i, b):
        pltpu.async_copy(table_hbm.at[idx_all.at[i]], rows[b], gsem[b])

    def gather_wait(b):
        pltpu.make_async_copy(table_hbm.at[idx_all.at[0]], rows[b],
                              gsem[b]).wait()

    def out_start(i, b):
        pltpu.async_copy(
            rows[b], out_hbm.at[pl.ds(i * nbatch + obase, BLOCK)], osem[b])

    def out_wait(b):
        pltpu.make_async_copy(
            rows[b], out_hbm.at[pl.ds(obase, BLOCK)], osem[b]).wait()

    gather_start(0, 0)
    gather_start(1, 1)
    gather_wait(0)
    out_start(0, 0)

    def pair(k, _):
        for b in (0, 1):
            i = 2 + 2 * k + b
            out_wait(b)
            gather_start(i, b)
            gather_wait(1 - b)
            out_start(i - 1, 1 - b)
        return ()

    lax.fori_loop(0, (n_chunks - 2) // 2, pair, (), unroll=False)

    gather_wait(1)
    out_start(n_chunks - 1, 1)
    out_wait(0)
    out_wait(1)


def kernel(indices, weight):
    b, s = indices.shape
    v, d = weight.shape
    n = b * s
    idx_perm = (indices.astype(jnp.int32)
                .reshape(NUM_WORKERS, BLOCK, s)
                .transpose(0, 2, 1))

    mesh = plsc.VectorSubcoreMesh(
        core_axis_name="c", subcore_axis_name="s",
        num_cores=NUM_CORES, num_subcores=NUM_SUBCORES,
    )
    run = functools.partial(
        pl.kernel,
        out_type=jax.ShapeDtypeStruct((n, d), jnp.float32),
        mesh=mesh,
        scratch_types=[
            pltpu.VMEM((s, BLOCK), jnp.int32),
            pltpu.VMEM((BLOCK, d), jnp.float32),
            pltpu.VMEM((BLOCK, d), jnp.float32),
            pltpu.SemaphoreType.DMA,
            pltpu.SemaphoreType.DMA,
            pltpu.SemaphoreType.DMA,
            pltpu.SemaphoreType.DMA,
        ],
    )(_gather_body)
    out = run(idx_perm, weight)
    return out.reshape(s, b, d).transpose(1, 0, 2)


# speedup vs baseline: 10.6224x; 1.0204x over previous
import functools

import jax
import jax.numpy as jnp
from jax import lax
from jax.experimental import pallas as pl
from jax.experimental.pallas import tpu as pltpu
from jax.experimental.pallas import tpu_sc as plsc

NUM_CORES = 2
NUM_SUBCORES = 16
NUM_WORKERS = NUM_CORES * NUM_SUBCORES
STREAM = 128
WBATCH = 256
NBLK = 16
NHALF = 2


def _gather_body(idx_hbm, table_hbm, out_hbm, idx_all, rows0, rows1,
                 g0, g1, o0, o1):
    n_slots = idx_hbm.shape[2] // (WBATCH // STREAM)
    n_seq = n_slots * NHALF
    wid = lax.axis_index("s") * NUM_CORES + lax.axis_index("c")
    p = wid // NHALF
    h = wid % NHALF
    nbatch = out_hbm.shape[0] // n_seq
    obase = h * n_slots * nbatch + p * WBATCH

    rows = (rows0, rows1)
    gsem = (g0, g1)
    osem = (o0, o1)

    pltpu.sync_copy(idx_hbm.at[p, h], idx_all)

    def gathers_start(i, b):
        for j in range(WBATCH // STREAM):
            pltpu.async_copy(
                table_hbm.at[idx_all.at[i * (WBATCH // STREAM) + j]],
                rows[b].at[pl.ds(j * STREAM, STREAM)], gsem[b])

    def gathers_wait(b):
        for j in range(WBATCH // STREAM):
            pltpu.make_async_copy(
                table_hbm.at[idx_all.at[0]],
                rows[b].at[pl.ds(j * STREAM, STREAM)], gsem[b]).wait()

    def out_start(i, b):
        pltpu.async_copy(
            rows[b], out_hbm.at[pl.ds(i * nbatch + obase, WBATCH)], osem[b])

    def out_wait(b):
        pltpu.make_async_copy(
            rows[b], out_hbm.at[pl.ds(obase, WBATCH)], osem[b]).wait()

    gathers_start(0, 0)
    gathers_start(1, 1)
    gathers_wait(0)
    out_start(0, 0)

    def pair(k, _):
        for b in (0, 1):
            i = 2 + 2 * k + b
            out_wait(b)
            gathers_start(i, b)
            gathers_wait(1 - b)
            out_start(i - 1, 1 - b)
        return ()

    n_steady = n_slots - 2
    lax.fori_loop(0, n_steady // 2, pair, (), unroll=False)
    if n_steady % 2:
        i = n_slots - 1
        b = i % 2
        out_wait(b)
        gathers_start(i, b)
        gathers_wait(1 - b)
        out_start(i - 1, 1 - b)

    last = n_slots - 1
    gathers_wait(last % 2)
    out_start(last, last % 2)
    out_wait((last + 1) % 2)
    out_wait(last % 2)


def kernel(indices, weight):
    b, s = indices.shape
    v, d = weight.shape
    n = b * s
    idx_perm = (indices.astype(jnp.int32)
                .reshape(NBLK, WBATCH // STREAM, STREAM, s)
                .transpose(0, 3, 1, 2)
                .reshape(NBLK, NHALF, (s // NHALF) * (WBATCH // STREAM),
                         STREAM))

    mesh = plsc.VectorSubcoreMesh(
        core_axis_name="c", subcore_axis_name="s",
        num_cores=NUM_CORES, num_subcores=NUM_SUBCORES,
    )
    run = functools.partial(
        pl.kernel,
        out_type=jax.ShapeDtypeStruct((n, d), jnp.float32),
        mesh=mesh,
        scratch_types=[
            pltpu.VMEM(((s // NHALF) * (WBATCH // STREAM), STREAM),
                       jnp.int32),
            pltpu.VMEM((WBATCH, d), jnp.float32),
            pltpu.VMEM((WBATCH, d), jnp.float32),
            pltpu.SemaphoreType.DMA,
            pltpu.SemaphoreType.DMA,
            pltpu.SemaphoreType.DMA,
            pltpu.SemaphoreType.DMA,
        ],
    )(_gather_body)
    out = run(idx_perm, weight)
    return out.reshape(s, b, d).transpose(1, 0, 2)
